# baseline (speedup 1.0000x reference)
import jax
import jax.numpy as jnp
from jax.experimental import pallas as pl

B = 32
OBS = 72
SEQ = 96
N = 21
NP_ = 24
DM = 128
DFF = 256
L = 2
E = 4
EP = 8
K = 2
LPRED = 96
RT = NP_ * SEQ
RD = LPRED * NP_

_F = jnp.float32
_HI = jax.lax.Precision.HIGHEST


def _moe_dec_kernel(x_ref, tt_ref, wstart_ref, bstart_ref, gw_ref, gb_ref,
                    w1_ref, b1_ref, w2_ref, b2_ref,
                    sw_ref, sb_ref, pw_ref, pb_ref,
                    dw1a_ref, dw1b_ref, db1_ref, dw2_ref, db2_ref,
                    dw3_ref, db3_ref,
                    out_ref):
    x = x_ref[0]
    tok = x * wstart_ref[...] + bstart_ref[...]

    for l in range(L):
        logits = jnp.dot(tok, gw_ref[l], preferred_element_type=_F,
                         precision=_HI) + gb_ref[...]
        lane = jax.lax.broadcasted_iota(jnp.int32, (RT, EP), 1)
        m1 = jnp.max(logits, axis=1, keepdims=True)
        i1 = jnp.min(jnp.where(logits == m1, lane, EP), axis=1, keepdims=True)
        is1 = lane == i1
        l2 = jnp.where(is1, -1e30, logits)
        m2 = jnp.max(l2, axis=1, keepdims=True)
        i2 = jnp.min(jnp.where(l2 == m2, lane, EP), axis=1, keepdims=True)
        is2 = lane == i2
        g1 = 1.0 / (1.0 + jnp.exp(m2 - m1))
        gates = g1 * is1.astype(_F) + (1.0 - g1) * is2.astype(_F)

        h = jnp.maximum(jnp.dot(tok, w1_ref[l], preferred_element_type=_F)
                        + b1_ref[l], 0.0)
        gbig = jnp.concatenate(
            [jnp.broadcast_to(gates[:, e:e + 1], (RT, DFF)) for e in range(E)],
            axis=1)
        y = jnp.dot(h * gbig, w2_ref[l], preferred_element_type=_F)
        for e in range(E):
            y = y + gates[:, e:e + 1] * b2_ref[l, e]
        tok = tok + y

    enc = jnp.sum(tok.reshape(NP_, SEQ, DM), axis=1) * (1.0 / SEQ)
    a = jnp.dot(enc, dw1a_ref[...], preferred_element_type=_F,
                precision=_HI)

    tt = tt_ref[0]
    lane = jax.lax.broadcasted_iota(jnp.int32, (LPRED, DM), 1)
    te = jnp.where(lane == 0, tt * sw_ref[...] + sb_ref[...],
                   jnp.sin(tt * pw_ref[...] + pb_ref[...]))
    c = jnp.dot(te, dw1b_ref[...], preferred_element_type=_F,
                precision=_HI)

    h1 = (c[:, None, :] + a[None, :, :] + db1_ref[...]).reshape(RD, DM)
    h1 = jnp.maximum(h1, 0.0)
    h2 = jnp.maximum(jnp.dot(h1, dw2_ref[...], preferred_element_type=_F)
                     + db2_ref[...], 0.0)
    o = jnp.dot(h2, dw3_ref[...], preferred_element_type=_F,
                precision=_HI) + db3_ref[...]
    out_ref[0] = o


def kernel(tp_to_predict, observed_data, observed_tp, observed_mask, W_start,
           b_start, gate_W, e_W1, e_b1, e_W2, e_b2, te_scale_W, te_scale_b,
           te_per_W, te_per_b, dec_W1, dec_b1, dec_W2, dec_b2, dec_W3, dec_b3):
    x = jnp.pad(observed_data, ((0, 0), (0, SEQ - OBS), (0, 0)))
    x = jnp.pad(x.transpose(0, 2, 1), ((0, 0), (0, NP_ - N), (0, 0)))
    x = x.reshape(B, RT, 1)
    tt = tp_to_predict.reshape(B, LPRED, 1)

    gw = jnp.concatenate([gate_W, jnp.zeros((L, DM, EP - E), _F)], axis=2)
    gb = jnp.concatenate([jnp.zeros((1, E), _F),
                          jnp.full((1, EP - E), -1e30, _F)], axis=1)
    w1 = e_W1.transpose(0, 2, 1, 3).reshape(L, DM, E * DFF)
    b1 = e_b1.reshape(L, 1, E * DFF)
    w2 = e_W2.reshape(L, E * DFF, DM)
    b2 = e_b2.reshape(L, E, 1, DM)

    sw = te_scale_W.reshape(1, 1)
    sb = te_scale_b.reshape(1, 1)
    pw = jnp.concatenate([jnp.zeros((1, 1), _F), te_per_W], axis=1)
    pb = jnp.concatenate([jnp.zeros((1, 1), _F),
                          te_per_b.reshape(1, DM - 1)], axis=1)

    dw1a = dec_W1[:DM]
    dw1b = dec_W1[DM:]
    db1 = dec_b1.reshape(1, DM)
    db2 = dec_b2.reshape(1, DM)
    db3 = dec_b3.reshape(1, 1)

    def full(shape):
        return pl.BlockSpec(shape, lambda i: (0,) * len(shape))

    o = pl.pallas_call(
        _moe_dec_kernel,
        grid=(B,),
        in_specs=[
            pl.BlockSpec((1, RT, 1), lambda i: (i, 0, 0)),
            pl.BlockSpec((1, LPRED, 1), lambda i: (i, 0, 0)),
            full((1, DM)), full((1, DM)),
            full((L, DM, EP)), full((1, EP)),
            full((L, DM, E * DFF)), full((L, 1, E * DFF)),
            full((L, E * DFF, DM)), full((L, E, 1, DM)),
            full((1, 1)), full((1, 1)), full((1, DM)), full((1, DM)),
            full((DM, DM)), full((DM, DM)), full((1, DM)),
            full((DM, DM)), full((1, DM)),
            full((DM, 1)), full((1, 1)),
        ],
        out_specs=pl.BlockSpec((1, RD, 1), lambda i: (i, 0, 0)),
        out_shape=jax.ShapeDtypeStruct((B, RD, 1), _F),
    )(x, tt, W_start, b_start.reshape(1, DM), gw, gb, w1, b1, w2, b2,
      sw, sb, pw, pb, dw1a, dw1b, db1, dec_W2, db2, dec_W3, db3)

    return o.reshape(B, LPRED, NP_)[:, :, :N][None]


# speedup vs baseline: 2.3476x; 2.3476x over previous
import numpy as np
import jax
import jax.numpy as jnp
from jax.experimental import pallas as pl

B = 32
OBS = 72
SEQ = 96
N = 21
NP_ = 24
DM = 128
DFF = 256
L = 2
E = 4
EP = 8
K = 2
LPRED = 96
RT = NP_ * SEQ
RD = LPRED * NP_

_F = jnp.float32
_BF = jnp.bfloat16
_HI = jax.lax.Precision.HIGHEST


def _dot_exact(a, b):
    ah = a.astype(_BF).astype(_F)
    al = a - ah
    return (jnp.dot(ah, b, preferred_element_type=_F)
            + jnp.dot(al, b, preferred_element_type=_F))


def _moe_dec_kernel(x_ref, tt_ref, wstart_ref, bstart_ref, gw_ref, gb_ref,
                    w1_ref, b1_ref, w2_ref, b2_ref,
                    sw_ref, sb_ref, pw_ref, pb_ref,
                    dw1a_ref, dw1b_ref, db1_ref, dw2_ref, db2_ref,
                    dw3_ref, db3_ref,
                    msum_ref, p1_ref, p2_ref,
                    out_ref):
    x = x_ref[0]
    tok = wstart_ref[...] * x + bstart_ref[...]

    for l in range(L):
        logits = jnp.dot(gw_ref[l], tok, preferred_element_type=_F,
                         precision=_HI) + gb_ref[...]
        sub = jax.lax.broadcasted_iota(jnp.int32, (EP, RT), 0)
        m1 = jnp.max(logits, axis=0, keepdims=True)
        i1 = jnp.min(jnp.where(logits == m1, sub, EP), axis=0, keepdims=True)
        is1 = sub == i1
        l2 = jnp.where(is1, -1e30, logits)
        m2 = jnp.max(l2, axis=0, keepdims=True)
        i2 = jnp.min(jnp.where(l2 == m2, sub, EP), axis=0, keepdims=True)
        is2 = sub == i2
        g1 = 1.0 / (1.0 + jnp.exp(m2 - m1))
        gates = g1 * is1.astype(_F) + (1.0 - g1) * is2.astype(_F)

        h = jnp.maximum(jnp.dot(w1_ref[l], tok, preferred_element_type=_F)
                        + b1_ref[l], 0.0)
        y = tok
        for e in range(E):
            ye = jnp.dot(w2_ref[l, e], h[e * DFF:(e + 1) * DFF],
                         preferred_element_type=_F) + b2_ref[l, e]
            y = y + gates[e:e + 1] * ye
        tok = y

    enc = _dot_exact(tok, msum_ref[...]) * (1.0 / SEQ)
    a = jnp.dot(dw1a_ref[...], enc, preferred_element_type=_F,
                precision=_HI)

    tt = tt_ref[0]
    sub = jax.lax.broadcasted_iota(jnp.int32, (DM, LPRED), 0)
    te = jnp.where(sub == 0, tt * sw_ref[...] + sb_ref[...],
                   jnp.sin(tt * pw_ref[...] + pb_ref[...]))
    c = jnp.dot(dw1b_ref[...], te, preferred_element_type=_F,
                precision=_HI)

    h1 = jnp.maximum(_dot_exact(c, p1_ref[...]) + _dot_exact(a, p2_ref[...])
                     + db1_ref[...], 0.0)
    h2 = jnp.maximum(jnp.dot(dw2_ref[...], h1, preferred_element_type=_F)
                     + db2_ref[...], 0.0)
    o = jnp.dot(dw3_ref[...], h2, preferred_element_type=_F,
                precision=_HI) + db3_ref[...]
    out_ref[0] = o


def kernel(tp_to_predict, observed_data, observed_tp, observed_mask, W_start,
           b_start, gate_W, e_W1, e_b1, e_W2, e_b2, te_scale_W, te_scale_b,
           te_per_W, te_per_b, dec_W1, dec_b1, dec_W2, dec_b2, dec_W3, dec_b3):
    x = jnp.pad(observed_data, ((0, 0), (0, SEQ - OBS), (0, 0)))
    x = jnp.pad(x.transpose(0, 2, 1), ((0, 0), (0, NP_ - N), (0, 0)))
    x = x.reshape(B, 1, RT)
    tt = tp_to_predict.reshape(B, 1, LPRED)

    gw = jnp.concatenate([gate_W.transpose(0, 2, 1),
                          jnp.zeros((L, EP - E, DM), _F)], axis=1)
    gb = jnp.concatenate([jnp.zeros((E, 1), _F),
                          jnp.full((EP - E, 1), -1e30, _F)], axis=0)
    w1 = e_W1.transpose(0, 1, 3, 2).reshape(L, E * DFF, DM)
    b1 = e_b1.reshape(L, E * DFF, 1)
    w2 = e_W2.transpose(0, 1, 3, 2)
    b2 = e_b2.reshape(L, E, DM, 1)

    sw = te_scale_W.reshape(1, 1)
    sb = te_scale_b.reshape(1, 1)
    pw = jnp.concatenate([jnp.zeros((1, 1), _F), te_per_W], axis=1).T
    pb = jnp.concatenate([jnp.zeros((1, 1), _F),
                          te_per_b.reshape(1, DM - 1)], axis=1).T

    dw1a = dec_W1[:DM].T
    dw1b = dec_W1[DM:].T
    db1 = dec_b1.reshape(DM, 1)
    dw2 = dec_W2.T
    db2 = dec_b2.reshape(DM, 1)
    dw3 = dec_W3.T
    db3 = dec_b3.reshape(1, 1)

    msum = jnp.asarray(np.kron(np.eye(NP_), np.ones((SEQ, 1))), _F)
    p1 = jnp.asarray(np.kron(np.eye(LPRED), np.ones((1, NP_))), _F)
    p2 = jnp.asarray(np.tile(np.eye(NP_), (1, LPRED)), _F)

    def full(shape):
        return pl.BlockSpec(shape, lambda i: (0,) * len(shape))

    o = pl.pallas_call(
        _moe_dec_kernel,
        grid=(B,),
        in_specs=[
            pl.BlockSpec((1, 1, RT), lambda i: (i, 0, 0)),
            pl.BlockSpec((1, 1, LPRED), lambda i: (i, 0, 0)),
            full((DM, 1)), full((DM, 1)),
            full((L, EP, DM)), full((EP, 1)),
            full((L, E * DFF, DM)), full((L, E * DFF, 1)),
            full((L, E, DM, DFF)), full((L, E, DM, 1)),
            full((1, 1)), full((1, 1)), full((DM, 1)), full((DM, 1)),
            full((DM, DM)), full((DM, DM)), full((DM, 1)),
            full((DM, DM)), full((DM, 1)),
            full((1, DM)), full((1, 1)),
            full((RT, NP_)), full((LPRED, RD)), full((NP_, RD)),
        ],
        out_specs=pl.BlockSpec((1, 1, RD), lambda i: (i, 0, 0)),
        out_shape=jax.ShapeDtypeStruct((B, 1, RD), _F),
    )(x, tt, W_start.reshape(DM, 1), b_start.reshape(DM, 1), gw, gb, w1, b1,
      w2, b2, sw, sb, pw, pb, dw1a, dw1b, db1, dw2, db2, dw3, db3,
      msum, p1, p2)

    return o.reshape(B, LPRED, NP_)[:, :, :N][None]
